# Initial kernel scaffold; baseline (speedup 1.0000x reference)
import jax
import jax.numpy as jnp
from jax.experimental import pallas as pl


def kernel(t, x, adj_norm, w1, b1, ln_g, ln_b, w2, b2, attn_w, attn_b, diffusion_scale, time_scale):
    raise NotImplementedError("write your pallas kernel here")


# speedup vs baseline: 1.9869x; 1.9869x over previous
import functools

import jax
import jax.numpy as jnp
from jax.experimental import pallas as pl
from jax.experimental.pallas import tpu as pltpu


def _fused_body(adj_ref, x_ref, w1t_ref, b1_ref, g_ref, be_ref, w2t_ref,
                b2_ref, awrep_ref, ds_ref, ts_ref, out_ref, xw_ref,
                *, B, N, D, BI):
    i = pl.program_id(0)

    @pl.when(i == 0)
    def _init():
        ds = ds_ref[0, 0]
        for b in range(B):
            xb = x_ref[b]
            L = jnp.dot(xb, awrep_ref[...],
                        preferred_element_type=jnp.float32)
            m = jnp.max(L, axis=0, keepdims=True)
            E = jnp.exp(L - m)
            Z = jnp.sum(E, axis=0, keepdims=True)
            xw_ref[:, b * D:(b + 1) * D] = xb * (E * (ds / Z))

    diff = jnp.dot(adj_ref[...], xw_ref[...],
                   preferred_element_type=jnp.float32)

    xi = x_ref[:, pl.ds(i * BI, BI), :].reshape(B * BI, D)
    h = jnp.dot(xi, w1t_ref[...], preferred_element_type=jnp.float32)
    h = h + b1_ref[...]
    mu = jnp.mean(h, axis=-1, keepdims=True)
    hc = h - mu
    var = jnp.mean(hc * hc, axis=-1, keepdims=True)
    h = hc * jax.lax.rsqrt(var + 1e-5) * g_ref[...] + be_ref[...]
    h = h * jax.nn.sigmoid(h)
    h = jnp.dot(h, w2t_ref[...], preferred_element_type=jnp.float32)
    dyn = jnp.tanh(h + b2_ref[...])

    ts = ts_ref[0, 0]
    for b in range(B):
        dx = ts * (dyn[b * BI:(b + 1) * BI, :] + diff[:, b * D:(b + 1) * D])
        nsq = jnp.sum(dx * dx, axis=-1, keepdims=True)
        scale = jnp.minimum(10.0 / (jnp.sqrt(nsq) + 1e-8), 1.0)
        out_ref[b] = dx * scale


@functools.partial(jax.jit, static_argnames=("interpret",))
def _run(x, adj_norm, w1t, b1, ln_g, ln_b, w2t, b2, awrep, ds, ts,
         interpret=False):
    B, N, D = x.shape
    BI = 256 if N % 256 == 0 else N
    body = functools.partial(_fused_body, B=B, N=N, D=D, BI=BI)
    return pl.pallas_call(
        body,
        grid=(N // BI,),
        in_specs=[
            pl.BlockSpec((BI, N), lambda i: (i, 0)),
            pl.BlockSpec((B, N, D), lambda i: (0, 0, 0)),
            pl.BlockSpec((D, D), lambda i: (0, 0)),
            pl.BlockSpec((1, D), lambda i: (0, 0)),
            pl.BlockSpec((1, D), lambda i: (0, 0)),
            pl.BlockSpec((1, D), lambda i: (0, 0)),
            pl.BlockSpec((D, D), lambda i: (0, 0)),
            pl.BlockSpec((1, D), lambda i: (0, 0)),
            pl.BlockSpec((D, 128), lambda i: (0, 0)),
            pl.BlockSpec((1, 1), lambda i: (0, 0)),
            pl.BlockSpec((1, 1), lambda i: (0, 0)),
        ],
        out_specs=pl.BlockSpec((B, BI, D), lambda i: (0, i, 0)),
        out_shape=jax.ShapeDtypeStruct((B, N, D), jnp.float32),
        scratch_shapes=[pltpu.VMEM((N, B * D), jnp.float32)],
        compiler_params=pltpu.CompilerParams(
            vmem_limit_bytes=60 * 1024 * 1024),
        interpret=interpret,
    )(adj_norm, x, w1t, b1, ln_g, ln_b, w2t, b2, awrep, ds, ts)


def kernel(t, x, adj_norm, w1, b1, ln_g, ln_b, w2, b2, attn_w, attn_b,
           diffusion_scale, time_scale, interpret=False):
    D = x.shape[-1]
    return _run(x, adj_norm, w1.T, b1.reshape(1, D), ln_g.reshape(1, D),
                ln_b.reshape(1, D), w2.T, b2.reshape(1, D),
                jnp.broadcast_to(attn_w.reshape(D, 1), (D, 128)),
                diffusion_scale.reshape(1, 1), time_scale.reshape(1, 1),
                interpret=interpret)
